# Initial kernel scaffold; baseline (speedup 1.0000x reference)
import jax
import jax.numpy as jnp
from jax.experimental import pallas as pl


def kernel(T, ebdx_w, ebdy_w):
    raise NotImplementedError("write your pallas kernel here")


# speedup vs baseline: 5.0970x; 5.0970x over previous
import functools

import jax
import jax.numpy as jnp
from jax import lax
from jax.experimental import pallas as pl
from jax.experimental.pallas import tpu as pltpu
from jax.experimental.pallas import tpu_sc as plsc

NX, NY = 1000, 1000
DIM = 64
XMIN, XMAX, YMIN, YMAX = 0.0, 1.0, 0.0, 1.0
DX = (XMAX - XMIN) / NX
DY = (YMAX - YMIN) / NY

L = 16
CB = 128


@functools.lru_cache(maxsize=None)
def _build(n_points: int):
    info = plsc.get_sparse_core_info()
    nc, ns = info.num_cores, info.num_subcores
    nw = nc * ns
    npw = n_points // nw
    nblk = npw // CB
    assert npw * nw == n_points and nblk * CB == npw

    mesh = plsc.VectorSubcoreMesh(core_axis_name="c", subcore_axis_name="s")

    @functools.partial(
        pl.kernel,
        out_type=jax.ShapeDtypeStruct((n_points, DIM), jnp.float32),
        mesh=mesh,
        scratch_types=[
            pltpu.VMEM((CB,), jnp.float32),
            pltpu.VMEM((CB,), jnp.float32),
            pltpu.VMEM((CB,), jnp.int32),
            pltpu.VMEM((CB,), jnp.int32),
            pltpu.VMEM((CB, 2 * DIM), jnp.float32),
            pltpu.VMEM((CB, 2 * DIM), jnp.float32),
            pltpu.VMEM((CB, DIM), jnp.float32),
            pltpu.SemaphoreType.DMA,
            pltpu.SemaphoreType.DMA,
        ],
    )
    def lookup(xs_hbm, ys_hbm, comb_hbm, out_hbm,
               xv_v, yv_v, idxx_v, idxy_v, bufx_v, bufy_v, outb_v, semx, semy):
        wid = lax.axis_index("s") * nc + lax.axis_index("c")
        wbase = wid * npw

        def block_body(blk, carry):
            base = pl.multiple_of(wbase + blk * CB, CB)
            pltpu.sync_copy(xs_hbm.at[pl.ds(base, CB)], xv_v)
            pltpu.sync_copy(ys_hbm.at[pl.ds(base, CB)], yv_v)

            def idx_step(j, c):
                s = pl.ds(j * 16, 16)
                x = xv_v[s]
                y = yv_v[s]
                xi = ((x - XMIN) / DX).astype(jnp.int32)
                yi = ((y - YMIN) / DY).astype(jnp.int32)
                xi = jnp.where((xi > NX) | (xi < 0), NX, xi)
                yi = jnp.where((yi > NY) | (yi < 0), NY, yi)
                idxx_v[s] = xi
                idxy_v[s] = yi
                return c
            lax.fori_loop(0, CB // 16, idx_step, 0, unroll=True)

            cx = pltpu.async_copy(comb_hbm.at[idxx_v], bufx_v, semx)
            cy = pltpu.async_copy(comb_hbm.at[idxy_v], bufy_v, semy)
            cx.wait()
            cy.wait()

            def add_row(i, c):
                for col in range(DIM // 16):
                    sa = pl.ds(col * 16, 16)
                    sb = pl.ds(DIM + col * 16, 16)
                    outb_v[i, sa] = bufx_v[i, sa] + bufy_v[i, sb]
                return c
            lax.fori_loop(0, CB, add_row, 0)

            pltpu.sync_copy(outb_v, out_hbm.at[pl.ds(base, CB)])
            return carry

        lax.fori_loop(0, nblk, block_body, 0)

    return lookup


def kernel(T, ebdx_w, ebdy_w):
    b, h, _ = T.shape
    n = b * h
    xs = T[:, :, 0].reshape(n)
    ys = T[:, :, 1].reshape(n)
    comb = jnp.concatenate([ebdx_w, ebdy_w], axis=1)
    out = _build(n)(xs, ys, comb)
    return out.reshape(b, h, DIM)


# speedup vs baseline: 6.0515x; 1.1873x over previous
import functools

import jax
import jax.numpy as jnp
from jax import lax
from jax.experimental import pallas as pl
from jax.experimental.pallas import tpu as pltpu
from jax.experimental.pallas import tpu_sc as plsc

NX, NY = 1000, 1000
DIM = 64
XMIN, XMAX, YMIN, YMAX = 0.0, 1.0, 0.0, 1.0
DX = (XMAX - XMIN) / NX
DY = (YMAX - YMIN) / NY

L = 16
CB = 64
KPB = 2
SBP = CB * KPB


@functools.lru_cache(maxsize=None)
def _build(n_points: int):
    info = plsc.get_sparse_core_info()
    nc, ns = info.num_cores, info.num_subcores
    nw = nc * ns
    npw = n_points // nw
    nu = npw // SBP
    assert npw * nw == n_points and nu * SBP == npw and nu % 2 == 0

    mesh = plsc.VectorSubcoreMesh(core_axis_name="c", subcore_axis_name="s")

    @functools.partial(
        pl.kernel,
        out_type=jax.ShapeDtypeStruct((n_points, DIM), jnp.float32),
        mesh=mesh,
        scratch_types=[
            [pltpu.VMEM((SBP,), jnp.float32) for _ in range(2)],
            [pltpu.VMEM((SBP,), jnp.float32) for _ in range(2)],
            [[pltpu.VMEM((CB,), jnp.int32) for _ in range(KPB)]
             for _ in range(2)],
            [[pltpu.VMEM((CB,), jnp.int32) for _ in range(KPB)]
             for _ in range(2)],
            [[pltpu.VMEM((CB, 2 * DIM), jnp.float32) for _ in range(KPB)]
             for _ in range(2)],
            [[pltpu.VMEM((CB, 2 * DIM), jnp.float32) for _ in range(KPB)]
             for _ in range(2)],
            [pltpu.VMEM((SBP, DIM), jnp.float32) for _ in range(2)],
            [pltpu.SemaphoreType.DMA for _ in range(2)],
            [pltpu.SemaphoreType.DMA for _ in range(2)],
            [pltpu.SemaphoreType.DMA for _ in range(2)],
        ],
    )
    def lookup(xs_hbm, ys_hbm, comb_hbm, out_hbm,
               cxv, cyv, idxx, idxy, bufx, bufy, outb, semc, semg, semo):
        wid = lax.axis_index("s") * nc + lax.axis_index("c")
        wbase = wid * npw

        def fire_coords(u, cs):
            gb = wbase + u * SBP
            pltpu.async_copy(xs_hbm.at[pl.ds(gb, SBP)], cxv[cs], semc[cs])
            pltpu.async_copy(ys_hbm.at[pl.ds(gb, SBP)], cyv[cs], semc[cs])

        def wait_coords(cs):
            pltpu.make_async_copy(xs_hbm.at[pl.ds(0, SBP)], cxv[cs], semc[cs]).wait()
            pltpu.make_async_copy(ys_hbm.at[pl.ds(0, SBP)], cyv[cs], semc[cs]).wait()

        def front(s):
            for k in range(KPB):
                for j in range(CB // L):
                    c = pl.ds(k * CB + j * L, L)
                    x = cxv[s][c]
                    y = cyv[s][c]
                    xi = ((x - XMIN) / DX).astype(jnp.int32)
                    yi = ((y - YMIN) / DY).astype(jnp.int32)
                    xi = jnp.where((xi > NX) | (xi < 0), NX, xi)
                    yi = jnp.where((yi > NY) | (yi < 0), NY, yi)
                    d = pl.ds(j * L, L)
                    idxx[s][k][d] = xi
                    idxy[s][k][d] = yi
                pltpu.async_copy(comb_hbm.at[idxx[s][k]], bufx[s][k], semg[s])
                pltpu.async_copy(comb_hbm.at[idxy[s][k]], bufy[s][k], semg[s])

        def back(u, s):
            @pl.when(u >= 2)
            def _():
                pltpu.make_async_copy(
                    outb[s], out_hbm.at[pl.ds(0, SBP)], semo[s]).wait()
            for k in range(KPB):
                pltpu.make_async_copy(
                    comb_hbm.at[idxx[s][k]], bufx[s][k], semg[s]).wait()
                pltpu.make_async_copy(
                    comb_hbm.at[idxy[s][k]], bufy[s][k], semg[s]).wait()

            def add_row(i, c):
                for k in range(KPB):
                    for col in range(DIM // L):
                        sa = pl.ds(col * L, L)
                        sb = pl.ds(DIM + col * L, L)
                        outb[s][k * CB + i, sa] = (
                            bufx[s][k][i, sa] + bufy[s][k][i, sb])
                return c
            lax.fori_loop(0, CB, add_row, 0)
            pltpu.async_copy(
                outb[s], out_hbm.at[pl.ds(wbase + u * SBP, SBP)], semo[s])

        fire_coords(0, 0)

        def pair_body(q, carry):
            u0 = 2 * q
            u1 = u0 + 1
            wait_coords(0)
            fire_coords(u1, 1)
            front(0)

            @pl.when(q > 0)
            def _():
                back(u0 - 1, 1)

            wait_coords(1)

            @pl.when(u0 + 2 < nu)
            def _():
                fire_coords(u0 + 2, 0)
            front(1)
            back(u0, 0)
            return carry

        lax.fori_loop(0, nu // 2, pair_body, 0)
        back(nu - 1, 1)
        for s in range(2):
            pltpu.make_async_copy(
                outb[s], out_hbm.at[pl.ds(0, SBP)], semo[s]).wait()

    return lookup


def kernel(T, ebdx_w, ebdy_w):
    b, h, _ = T.shape
    n = b * h
    xs = T[:, :, 0].reshape(n)
    ys = T[:, :, 1].reshape(n)
    comb = jnp.concatenate([ebdx_w, ebdy_w], axis=1)
    out = _build(n)(xs, ys, comb)
    return out.reshape(b, h, DIM)
